# Initial kernel scaffold; baseline (speedup 1.0000x reference)
import jax
import jax.numpy as jnp
from jax.experimental import pallas as pl


def kernel(in_feats, sample_priors, sample_feats, map_hw, map_offs, map_ids, W_off, b_off, W_aw, b_aw, W_val, b_val, W_out, b_out):
    raise NotImplementedError("write your pallas kernel here")


# speedup vs baseline: 86.3688x; 86.3688x over previous
import functools
import math

import jax
import jax.numpy as jnp
import numpy as np
from jax import lax
from jax.experimental import pallas as pl
from jax.experimental.pallas import tpu as pltpu
from jax.experimental.pallas import tpu_sc as plsc

B, N, S = 2, 5440, 5440
H = 8
HD = 32
BN = 680
Q = 160


def _prep_body(inf_ref, sf_ref, prior_ref, mid_ref, Wbig_ref, bbig_ref,
               Wval_ref, bval_ref, G_ref, hw_ref, offs_ref,
               val_ref, idx_ref, w_ref):
    f32 = jnp.float32
    xin = inf_ref[...]
    sfx = sf_ref[...]

    val_ref[...] = jnp.dot(sfx, Wval_ref[...], preferred_element_type=f32) + bval_ref[...]

    big = jnp.dot(xin, Wbig_ref[...], preferred_element_type=f32) + bbig_ref[...]
    ox = big[:, 0:64]
    oy = big[:, 64:128]
    oz = big[:, 128:192]
    aw = big[:, 192:256]

    e = jnp.exp(aw)
    ssum = jnp.dot(e, G_ref[...], preferred_element_type=f32)
    attn = e / ssum

    mid = mid_ref[...]
    midf = mid.astype(f32)

    def sel4(lvl, t0, t1, t2, t3):
        r = jnp.where(lvl == 0, t0, jnp.where(lvl == 1, t1, jnp.where(lvl == 2, t2, t3)))
        return r

    hw = [(hw_ref[k, 0], hw_ref[k, 1]) for k in range(4)]
    offs_t = [offs_ref[k] for k in range(4)]

    normW = sel4(mid, hw[0][1], hw[1][1], hw[2][1], hw[3][1]).astype(f32)
    normH = sel4(mid, hw[0][0], hw[1][0], hw[2][0], hw[3][0]).astype(f32)
    x = prior_ref[:, 0:1] + ox / normW
    y = prior_ref[:, 1:2] + oy / normH

    pmod = (lax.broadcasted_iota(jnp.int32, (BN, 64), 1) % 2).astype(f32)
    zbase = midf + (pmod - 0.5)
    z = (zbase + jnp.tanh(oz)) / 3.0
    zc = jnp.clip(z, 0.0, 1.0) * 3.0
    z0f = jnp.clip(jnp.floor(zc), 0.0, 2.0)
    wz = jnp.clip(zc - z0f, 0.0, 1.0)
    z0 = z0f.astype(jnp.int32)

    idx_blocks = []
    w_blocks = []
    for L in (0, 1):
        lvl = z0 + L
        Hl = sel4(lvl, hw[0][0], hw[1][0], hw[2][0], hw[3][0]).astype(f32)
        Wl = sel4(lvl, hw[0][1], hw[1][1], hw[2][1], hw[3][1]).astype(f32)
        offv = sel4(lvl, offs_t[0], offs_t[1], offs_t[2], offs_t[3])
        h_im = y * Hl - 0.5
        w_im = x * Wl - 0.5
        h0 = jnp.floor(h_im)
        w0 = jnp.floor(w_im)
        lh = h_im - h0
        lw = w_im - w0
        h0i = h0.astype(jnp.int32)
        w0i = w0.astype(jnp.int32)
        Hli = Hl.astype(jnp.int32)
        Wli = Wl.astype(jnp.int32)
        lvlw = (1.0 - wz) if L == 0 else wz
        base_w = attn * lvlw
        for dh, dw, cw in ((0, 0, (1 - lh) * (1 - lw)), (0, 1, (1 - lh) * lw),
                           (1, 0, lh * (1 - lw)), (1, 1, lh * lw)):
            hh = h0i + dh
            ww = w0i + dw
            valid = (hh >= 0) & (hh < Hli) & (ww >= 0) & (ww < Wli)
            hs = jnp.clip(hh, 0, Hli - 1)
            ws = jnp.clip(ww, 0, Wli - 1)
            idx_blocks.append(offv + hs * Wli + ws)
            w_blocks.append(jnp.where(valid, base_w * cw, 0.0))

    idx_ref[...] = jnp.concatenate(idx_blocks, axis=1)
    w_ref[...] = jnp.concatenate(w_blocks, axis=1)


def _run_prep(inf, sf, prior, mid3, Wbig, bbig2, Wval, bval2, G, map_hw, map_offs):
    full = lambda shp: pl.BlockSpec(shp, lambda b, i: (0,) * len(shp))
    blk = lambda c: pl.BlockSpec((None, BN, c), lambda b, i: (b, i, 0))
    smem = lambda shp: pl.BlockSpec(shp, lambda b, i: (0,) * len(shp),
                                    memory_space=pltpu.SMEM)
    return pl.pallas_call(
        _prep_body,
        grid=(B, N // BN),
        in_specs=[blk(256), blk(256), blk(2), blk(1),
                  full((256, 256)), full((1, 256)),
                  full((256, 256)), full((1, 256)),
                  full((64, 64)), smem((4, 2)), smem((4,))],
        out_specs=[blk(256), blk(512), blk(512)],
        out_shape=[jax.ShapeDtypeStruct((B, N, 256), jnp.float32),
                   jax.ShapeDtypeStruct((B, N, 512), jnp.int32),
                   jax.ShapeDtypeStruct((B, N, 512), jnp.float32)],
    )(inf, sf, prior, mid3, Wbig, bbig2, Wval, bval2, G, map_hw, map_offs)


_COLS = None


def _sc_gather_call(val_r, idx_c, w_c):
    mesh = plsc.VectorSubcoreMesh(core_axis_name="c", subcore_axis_name="s")
    NG = N // Q

    @functools.partial(
        pl.kernel,
        out_type=jax.ShapeDtypeStruct((B, 16, NG, 16 * Q), jnp.float32),
        mesh=mesh,
        compiler_params=pltpu.CompilerParams(needs_layout_passes=False),
        scratch_types=[
            pltpu.VMEM((S * 16,), jnp.float32),
            pltpu.VMEM((64 * Q,), jnp.int32),
            pltpu.VMEM((64 * Q,), jnp.float32),
            pltpu.VMEM((16 * Q,), jnp.float32),
        ],
    )
    def sc_gather(val_hbm, idxt_hbm, wt_hbm, out_hbm, table_v, idx_v, w_v, ob_v):
        cid = lax.axis_index("c")
        sid = lax.axis_index("s")
        wid = sid * 2 + cid
        bh = wid % 16
        half = wid // 16
        b = bh // 8
        cg = (bh % 8) * 2 + half
        pltpu.sync_copy(val_hbm.at[b, cg], table_v)

        def chunk(gi, carry):
            pltpu.sync_copy(idxt_hbm.at[bh, gi], idx_v)
            pltpu.sync_copy(wt_hbm.at[bh, gi], w_v)

            def group(qg, c2):
                qo = pl.multiple_of(qg * 16, 16)

                def jbody(j, acc):
                    iv = idx_v[pl.ds(j * Q + qo, 16)]
                    wv = w_v[pl.ds(j * Q + qo, 16)]
                    iv16 = iv * 16
                    return tuple(
                        acc[d] + wv * plsc.load_gather(table_v, [iv16 + d])
                        for d in range(16))

                acc = lax.fori_loop(
                    0, 64, jbody,
                    tuple(jnp.zeros((16,), jnp.float32) for _ in range(16)))
                for d in range(16):
                    ob_v[pl.ds(d * Q + qo, 16)] = acc[d]
                return c2

            lax.fori_loop(0, Q // 16, group, 0)
            pltpu.sync_copy(ob_v, out_hbm.at[b, cg, gi])
            return carry

        lax.fori_loop(0, NG, chunk, 0)

    return sc_gather(val_r, idx_c, w_c)


def _outproj_body(at_ref, Wout_ref, bout_ref, out_ref):
    a = at_ref[...]
    o = lax.dot_general(a, Wout_ref[...], (((0,), (0,)), ((), ())),
                        preferred_element_type=jnp.float32)
    out_ref[...] = o + bout_ref[...]


def _run_outproj(at, Wout, bout2):
    full = lambda shp: pl.BlockSpec(shp, lambda b: (0,) * len(shp))
    return pl.pallas_call(
        _outproj_body,
        grid=(B,),
        in_specs=[pl.BlockSpec((None, 256, N), lambda b: (b, 0, 0)),
                  full((256, 256)), full((1, 256))],
        out_specs=pl.BlockSpec((None, N, 256), lambda b: (b, 0, 0)),
        out_shape=jax.ShapeDtypeStruct((B, N, 256), jnp.float32),
    )(at, Wout, bout2)


def kernel(in_feats, sample_priors, sample_feats, map_hw, map_offs, map_ids,
           W_off, b_off, W_aw, b_aw, W_val, b_val, W_out, b_out):
    f32 = jnp.float32
    Wo = W_off.reshape(256, H, 8, 3)
    bo = b_off.reshape(H, 8, 3)
    Wbig = jnp.concatenate([Wo[..., 0].reshape(256, 64), Wo[..., 1].reshape(256, 64),
                            Wo[..., 2].reshape(256, 64), W_aw], axis=1)
    bbig = jnp.concatenate([bo[..., 0].reshape(64), bo[..., 1].reshape(64),
                            bo[..., 2].reshape(64), b_aw])[None, :]
    G = jnp.asarray(np.kron(np.eye(8, dtype=np.float32),
                            np.ones((8, 8), dtype=np.float32)))

    val, idx_nat, w_nat = _run_prep(
        in_feats, sample_feats, sample_priors,
        map_ids[..., None].astype(jnp.int32), Wbig, bbig,
        W_val, b_val[None, :], G, map_hw, map_offs)

    NG = N // Q
    idx_c = (idx_nat.reshape(B, NG, Q, 8, 8, 8).transpose(0, 4, 1, 3, 5, 2)
             .reshape(B * H, NG, 64 * Q))
    w_c = (w_nat.reshape(B, NG, Q, 8, 8, 8).transpose(0, 4, 1, 3, 5, 2)
           .reshape(B * H, NG, 64 * Q))
    val_r = val.reshape(B, S, 16, 16).transpose(0, 2, 1, 3).reshape(B, 16, S * 16)

    at5 = _sc_gather_call(val_r, idx_c, w_c)
    at = (at5.reshape(B, 16, NG, 16, Q).transpose(0, 1, 3, 2, 4)
          .reshape(B, 256, N))

    return _run_outproj(at, W_out, b_out[None, :])


# speedup vs baseline: 95.6937x; 1.1080x over previous
import functools
import math

import jax
import jax.numpy as jnp
import numpy as np
from jax import lax
from jax.experimental import pallas as pl
from jax.experimental.pallas import tpu as pltpu
from jax.experimental.pallas import tpu_sc as plsc

B, N, S = 2, 5440, 5440
H = 8
HD = 32
Q = 160
NG = N // Q


def _prep_body(inT_ref, sf_ref, priorT_ref, midT_ref, WbigT_ref, bbigT_ref,
               Wval_ref, bval_ref, G_ref, hw_ref, offs_ref,
               val_ref, idx_ref, w_ref):
    f32 = jnp.float32
    xinT = inT_ref[...]
    sfx = sf_ref[...]

    val = jnp.dot(sfx, Wval_ref[...], preferred_element_type=f32) + bval_ref[...]
    for cg in range(16):
        val_ref[cg, :, :] = val[:, cg * 16:(cg + 1) * 16]

    big = jnp.dot(WbigT_ref[...], xinT, preferred_element_type=f32) + bbigT_ref[...]
    ox = big[0:64, :]
    oy = big[64:128, :]
    oz = big[128:192, :]
    aw = big[192:256, :]

    e = jnp.exp(aw)
    ssum = jnp.dot(G_ref[...], e, preferred_element_type=f32)
    attn = e / ssum

    mid = midT_ref[...]
    midf = mid.astype(f32)

    def sel4(lvl, t0, t1, t2, t3):
        return jnp.where(lvl == 0, t0, jnp.where(lvl == 1, t1,
                         jnp.where(lvl == 2, t2, t3)))

    hw = [(hw_ref[k, 0], hw_ref[k, 1]) for k in range(4)]
    offs_t = [offs_ref[k] for k in range(4)]

    normW = sel4(mid, hw[0][1], hw[1][1], hw[2][1], hw[3][1]).astype(f32)
    normH = sel4(mid, hw[0][0], hw[1][0], hw[2][0], hw[3][0]).astype(f32)
    x = priorT_ref[0:1, :] + ox / normW
    y = priorT_ref[1:2, :] + oy / normH

    pmod = (lax.broadcasted_iota(jnp.int32, (64, Q), 0) % 2).astype(f32)
    zbase = midf + (pmod - 0.5)
    z = (zbase + jnp.tanh(oz)) / 3.0
    zc = jnp.clip(z, 0.0, 1.0) * 3.0
    z0f = jnp.clip(jnp.floor(zc), 0.0, 2.0)
    wz = jnp.clip(zc - z0f, 0.0, 1.0)
    z0 = z0f.astype(jnp.int32)

    idx_blocks = []
    w_blocks = []
    for L in (0, 1):
        lvl = z0 + L
        Hl = sel4(lvl, hw[0][0], hw[1][0], hw[2][0], hw[3][0]).astype(f32)
        Wl = sel4(lvl, hw[0][1], hw[1][1], hw[2][1], hw[3][1]).astype(f32)
        offv = sel4(lvl, offs_t[0], offs_t[1], offs_t[2], offs_t[3])
        h_im = y * Hl - 0.5
        w_im = x * Wl - 0.5
        h0 = jnp.floor(h_im)
        w0 = jnp.floor(w_im)
        lh = h_im - h0
        lw = w_im - w0
        h0i = h0.astype(jnp.int32)
        w0i = w0.astype(jnp.int32)
        Hli = Hl.astype(jnp.int32)
        Wli = Wl.astype(jnp.int32)
        lvlw = (1.0 - wz) if L == 0 else wz
        base_w = attn * lvlw
        for dh, dw, cw in ((0, 0, (1 - lh) * (1 - lw)), (0, 1, (1 - lh) * lw),
                           (1, 0, lh * (1 - lw)), (1, 1, lh * lw)):
            hh = h0i + dh
            ww = w0i + dw
            valid = (hh >= 0) & (hh < Hli) & (ww >= 0) & (ww < Wli)
            hs = jnp.clip(hh, 0, Hli - 1)
            ws = jnp.clip(ww, 0, Wli - 1)
            idx_blocks.append((offv + hs * Wli + ws) * 16)
            w_blocks.append(jnp.where(valid, base_w * cw, 0.0))

    for h in range(8):
        for cb in range(8):
            idx_ref[h, cb * 8:(cb + 1) * 8, :] = idx_blocks[cb][h * 8:(h + 1) * 8, :]
            w_ref[h, cb * 8:(cb + 1) * 8, :] = w_blocks[cb][h * 8:(h + 1) * 8, :]


def _run_prep(inT, sf, priorT, midT, WbigT, bbigT, Wval, bval2, G, map_hw, map_offs):
    full = lambda shp: pl.BlockSpec(shp, lambda b, i: (0,) * len(shp))
    smem = lambda shp: pl.BlockSpec(shp, lambda b, i: (0,) * len(shp),
                                    memory_space=pltpu.SMEM)
    return pl.pallas_call(
        _prep_body,
        grid=(B, NG),
        in_specs=[pl.BlockSpec((None, None, 256, Q), lambda b, i: (b, i, 0, 0)),
                  pl.BlockSpec((None, Q, 256), lambda b, i: (b, i, 0)),
                  pl.BlockSpec((None, None, 2, Q), lambda b, i: (b, i, 0, 0)),
                  pl.BlockSpec((None, None, 1, Q), lambda b, i: (b, i, 0, 0)),
                  full((256, 256)), full((256, 1)),
                  full((256, 256)), full((1, 256)),
                  full((64, 64)), smem((4, 2)), smem((4,))],
        out_specs=[pl.BlockSpec((None, 16, Q, 16), lambda b, i: (b, 0, i, 0)),
                   pl.BlockSpec((8, None, 64, Q), lambda b, i: (b, i, 0, 0)),
                   pl.BlockSpec((8, None, 64, Q), lambda b, i: (b, i, 0, 0))],
        out_shape=[jax.ShapeDtypeStruct((B, 16, S, 16), jnp.float32),
                   jax.ShapeDtypeStruct((2 * H, NG, 64, Q), jnp.int32),
                   jax.ShapeDtypeStruct((2 * H, NG, 64, Q), jnp.float32)],
    )(inT, sf, priorT, midT, WbigT, bbigT, Wval, bval2, G, map_hw, map_offs)


def _sc_gather_call(val_r, idx_c, w_c):
    mesh = plsc.VectorSubcoreMesh(core_axis_name="c", subcore_axis_name="s")

    @functools.partial(
        pl.kernel,
        out_type=jax.ShapeDtypeStruct((B, NG, 16, 16 * Q), jnp.float32),
        mesh=mesh,
        compiler_params=pltpu.CompilerParams(needs_layout_passes=False),
        scratch_types=[
            pltpu.VMEM((S * 16,), jnp.float32),
            pltpu.VMEM((64 * Q,), jnp.int32),
            pltpu.VMEM((64 * Q,), jnp.float32),
            pltpu.VMEM((16 * Q,), jnp.float32),
        ],
    )
    def sc_gather(val_hbm, idxt_hbm, wt_hbm, out_hbm, table_v, idx_v, w_v, ob_v):
        cid = lax.axis_index("c")
        sid = lax.axis_index("s")
        wid = sid * 2 + cid
        bh = wid % 16
        half = wid // 16
        b = bh // 8
        cg = (bh % 8) * 2 + half
        pltpu.sync_copy(val_hbm.at[b, cg], table_v)

        def chunk(gi, carry):
            pltpu.sync_copy(idxt_hbm.at[bh, gi], idx_v)
            pltpu.sync_copy(wt_hbm.at[bh, gi], w_v)

            def group(qg, c2):
                qo = pl.multiple_of(qg * 16, 16)

                def jbody(j, acc):
                    iv = idx_v[pl.ds(j * Q + qo, 16)]
                    wv = w_v[pl.ds(j * Q + qo, 16)]
                    return tuple(
                        acc[d] + wv * plsc.load_gather(table_v, [iv + d])
                        for d in range(16))

                acc = lax.fori_loop(
                    0, 64, jbody,
                    tuple(jnp.zeros((16,), jnp.float32) for _ in range(16)))
                for d in range(16):
                    ob_v[pl.ds(d * Q + qo, 16)] = acc[d]
                return c2

            lax.fori_loop(0, Q // 16, group, 0)
            pltpu.sync_copy(ob_v, out_hbm.at[b, gi, cg])
            return carry

        lax.fori_loop(0, NG, chunk, 0)

    return sc_gather(val_r, idx_c, w_c)


def _outproj_body(at_ref, Wout_ref, bout_ref, out_ref):
    a = at_ref[...].reshape(256, Q)
    o = lax.dot_general(a, Wout_ref[...], (((0,), (0,)), ((), ())),
                        preferred_element_type=jnp.float32)
    out_ref[...] = o + bout_ref[...]


def _run_outproj(at6, Wout, bout2):
    full = lambda shp: pl.BlockSpec(shp, lambda b, i: (0,) * len(shp))
    return pl.pallas_call(
        _outproj_body,
        grid=(B, NG),
        in_specs=[pl.BlockSpec((None, None, 16, 16, Q),
                               lambda b, i: (b, i, 0, 0, 0)),
                  full((256, 256)), full((1, 256))],
        out_specs=pl.BlockSpec((None, Q, 256), lambda b, i: (b, i, 0)),
        out_shape=jax.ShapeDtypeStruct((B, N, 256), jnp.float32),
    )(at6, Wout, bout2)


def kernel(in_feats, sample_priors, sample_feats, map_hw, map_offs, map_ids,
           W_off, b_off, W_aw, b_aw, W_val, b_val, W_out, b_out):
    Wo = W_off.reshape(256, H, 8, 3)
    bo = b_off.reshape(H, 8, 3)
    Wbig = jnp.concatenate([Wo[..., 0].reshape(256, 64), Wo[..., 1].reshape(256, 64),
                            Wo[..., 2].reshape(256, 64), W_aw], axis=1)
    bbig = jnp.concatenate([bo[..., 0].reshape(64), bo[..., 1].reshape(64),
                            bo[..., 2].reshape(64), b_aw])
    G = jnp.asarray(np.kron(np.eye(8, dtype=np.float32),
                            np.ones((8, 8), dtype=np.float32)))

    inT = in_feats.reshape(B, NG, Q, 256).transpose(0, 1, 3, 2)
    priorT = sample_priors.reshape(B, NG, Q, 2).transpose(0, 1, 3, 2)
    midT = map_ids.astype(jnp.int32).reshape(B, NG, 1, Q)

    val5, idx4, w4 = _run_prep(
        inT, sample_feats, priorT, midT, Wbig.T, bbig[:, None],
        W_val, b_val[None, :], G, map_hw, map_offs)

    val_r = val5.reshape(B, 16, S * 16)
    idx_c = idx4.reshape(2 * H, NG, 64 * Q)
    w_c = w4.reshape(2 * H, NG, 64 * Q)

    at5 = _sc_gather_call(val_r, idx_c, w_c)
    at6 = at5.reshape(B, NG, 16, 16, Q)

    return _run_outproj(at6, W_out, b_out[None, :])


# speedup vs baseline: 151.2232x; 1.5803x over previous
import functools
import math

import jax
import jax.numpy as jnp
import numpy as np
from jax import lax
from jax.experimental import pallas as pl
from jax.experimental.pallas import tpu as pltpu
from jax.experimental.pallas import tpu_sc as plsc

B, N, S = 2, 5440, 5440
H = 8
HD = 32
Q = 160
NG = N // Q
SP = 5504


def _prep_body(inT_ref, sf_ref, priorT_ref, midT_ref, WbigT_ref, bbigT_ref,
               Wval_ref, bval_ref, G_ref, hw_ref, offs_ref,
               val_ref, idx_ref, w_ref):
    f32 = jnp.float32
    xinT = inT_ref[...]
    sfx = sf_ref[...]

    val = jnp.dot(sfx, Wval_ref[...], preferred_element_type=f32) + bval_ref[...]
    for cg in range(16):
        val_ref[cg, :, :] = val[:, cg * 16:(cg + 1) * 16]

    big = jnp.dot(WbigT_ref[...], xinT, preferred_element_type=f32) + bbigT_ref[...]
    ox = big[0:64, :]
    oy = big[64:128, :]
    oz = big[128:192, :]
    aw = big[192:256, :]

    e = jnp.exp(aw)
    ssum = jnp.dot(G_ref[...], e, preferred_element_type=f32)
    attn = e / ssum

    mid = midT_ref[...]
    midf = mid.astype(f32)

    def sel4(lvl, t0, t1, t2, t3):
        return jnp.where(lvl == 0, t0, jnp.where(lvl == 1, t1,
                         jnp.where(lvl == 2, t2, t3)))

    hw = [(hw_ref[k, 0], hw_ref[k, 1]) for k in range(4)]
    offs_t = [offs_ref[k] for k in range(4)]

    normW = sel4(mid, hw[0][1], hw[1][1], hw[2][1], hw[3][1]).astype(f32)
    normH = sel4(mid, hw[0][0], hw[1][0], hw[2][0], hw[3][0]).astype(f32)
    x = priorT_ref[0:1, :] + ox / normW
    y = priorT_ref[1:2, :] + oy / normH

    pmod = (lax.broadcasted_iota(jnp.int32, (64, Q), 0) % 2).astype(f32)
    zbase = midf + (pmod - 0.5)
    z = (zbase + jnp.tanh(oz)) / 3.0
    zc = jnp.clip(z, 0.0, 1.0) * 3.0
    z0f = jnp.clip(jnp.floor(zc), 0.0, 2.0)
    wz = jnp.clip(zc - z0f, 0.0, 1.0)
    z0 = z0f.astype(jnp.int32)

    idx_blocks = []
    w_blocks = []
    for L in (0, 1):
        lvl = z0 + L
        Hl = sel4(lvl, hw[0][0], hw[1][0], hw[2][0], hw[3][0]).astype(f32)
        Wl = sel4(lvl, hw[0][1], hw[1][1], hw[2][1], hw[3][1]).astype(f32)
        offv = sel4(lvl, offs_t[0], offs_t[1], offs_t[2], offs_t[3])
        h_im = y * Hl - 0.5
        w_im = x * Wl - 0.5
        h0 = jnp.floor(h_im)
        w0 = jnp.floor(w_im)
        lh = h_im - h0
        lw = w_im - w0
        h0i = h0.astype(jnp.int32)
        w0i = w0.astype(jnp.int32)
        Hli = Hl.astype(jnp.int32)
        Wli = Wl.astype(jnp.int32)
        lvlw = (1.0 - wz) if L == 0 else wz
        base_w = attn * lvlw
        for dh, dw, cw in ((0, 0, (1 - lh) * (1 - lw)), (0, 1, (1 - lh) * lw),
                           (1, 0, lh * (1 - lw)), (1, 1, lh * lw)):
            hh = h0i + dh
            ww = w0i + dw
            valid = (hh >= 0) & (hh < Hli) & (ww >= 0) & (ww < Wli)
            hs = jnp.clip(hh, 0, Hli - 1)
            ws = jnp.clip(ww, 0, Wli - 1)
            idx_blocks.append(offv + hs * Wli + ws)
            w_blocks.append(jnp.where(valid, base_w * cw, 0.0))

    for h in range(8):
        for cb in range(8):
            idx_ref[h, cb * 8:(cb + 1) * 8, :] = idx_blocks[cb][h * 8:(h + 1) * 8, :]
            w_ref[h, cb * 8:(cb + 1) * 8, :] = w_blocks[cb][h * 8:(h + 1) * 8, :]


def _run_prep(inT, sf, priorT, midT, WbigT, bbigT, Wval, bval2, G, map_hw, map_offs):
    full = lambda shp: pl.BlockSpec(shp, lambda b, i: (0,) * len(shp))
    smem = lambda shp: pl.BlockSpec(shp, lambda b, i: (0,) * len(shp),
                                    memory_space=pltpu.SMEM)
    return pl.pallas_call(
        _prep_body,
        grid=(B, NG),
        in_specs=[pl.BlockSpec((None, None, 256, Q), lambda b, i: (b, i, 0, 0)),
                  pl.BlockSpec((None, Q, 256), lambda b, i: (b, i, 0)),
                  pl.BlockSpec((None, None, 2, Q), lambda b, i: (b, i, 0, 0)),
                  pl.BlockSpec((None, None, 1, Q), lambda b, i: (b, i, 0, 0)),
                  full((256, 256)), full((256, 1)),
                  full((256, 256)), full((1, 256)),
                  full((64, 64)), smem((4, 2)), smem((4,))],
        out_specs=[pl.BlockSpec((None, 16, Q, 16), lambda b, i: (b, 0, i, 0)),
                   pl.BlockSpec((8, None, 64, Q), lambda b, i: (b, i, 0, 0)),
                   pl.BlockSpec((8, None, 64, Q), lambda b, i: (b, i, 0, 0))],
        out_shape=[jax.ShapeDtypeStruct((B, 16, S, 16), jnp.float32),
                   jax.ShapeDtypeStruct((2 * H, NG, 64, Q), jnp.int32),
                   jax.ShapeDtypeStruct((2 * H, NG, 64, Q), jnp.float32)],
    )(inT, sf, priorT, midT, WbigT, bbigT, Wval, bval2, G, map_hw, map_offs)


def _sc_gather_call(val_r, idx_c, w_c):
    mesh = plsc.VectorSubcoreMesh(core_axis_name="c", subcore_axis_name="s")

    @functools.partial(
        pl.kernel,
        out_type=jax.ShapeDtypeStruct((B, NG, 16, 16 * Q), jnp.float32),
        mesh=mesh,
        compiler_params=pltpu.CompilerParams(needs_layout_passes=False),
        scratch_types=[
            pltpu.VMEM((16 * SP,), jnp.float32),
            pltpu.VMEM((64 * Q,), jnp.int32),
            pltpu.VMEM((64 * Q,), jnp.float32),
            pltpu.VMEM((16 * Q,), jnp.float32),
        ],
    )
    def sc_gather(val_hbm, idxt_hbm, wt_hbm, out_hbm, table_v, idx_v, w_v, ob_v):
        cid = lax.axis_index("c")
        sid = lax.axis_index("s")
        wid = sid * 2 + cid
        bh = wid % 16
        half = wid // 16
        b = bh // 8
        cg = (bh % 8) * 2 + half
        pltpu.sync_copy(val_hbm.at[b, cg], table_v)

        def chunk(gi, carry):
            pltpu.sync_copy(idxt_hbm.at[bh, gi], idx_v)
            pltpu.sync_copy(wt_hbm.at[bh, gi], w_v)

            def group(qg, c2):
                qo = pl.multiple_of(qg * 16, 16)

                def jbody(j, acc):
                    iv = idx_v[pl.ds(j * Q + qo, 16)]
                    wv = w_v[pl.ds(j * Q + qo, 16)]
                    return tuple(
                        acc[d] + wv * plsc.load_gather(
                            table_v.at[pl.ds(d * SP, S)], [iv])
                        for d in range(16))

                acc = lax.fori_loop(
                    0, 64, jbody,
                    tuple(jnp.zeros((16,), jnp.float32) for _ in range(16)))
                for d in range(16):
                    ob_v[pl.ds(d * Q + qo, 16)] = acc[d]
                return c2

            lax.fori_loop(0, Q // 16, group, 0)
            pltpu.sync_copy(ob_v, out_hbm.at[b, gi, cg])
            return carry

        lax.fori_loop(0, NG, chunk, 0)

    return sc_gather(val_r, idx_c, w_c)


def _outproj_body(at_ref, Wout_ref, bout_ref, out_ref):
    a = at_ref[...].reshape(256, Q)
    o = lax.dot_general(a, Wout_ref[...], (((0,), (0,)), ((), ())),
                        preferred_element_type=jnp.float32)
    out_ref[...] = o + bout_ref[...]


def _run_outproj(at6, Wout, bout2):
    full = lambda shp: pl.BlockSpec(shp, lambda b, i: (0,) * len(shp))
    return pl.pallas_call(
        _outproj_body,
        grid=(B, NG),
        in_specs=[pl.BlockSpec((None, None, 16, 16, Q),
                               lambda b, i: (b, i, 0, 0, 0)),
                  full((256, 256)), full((1, 256))],
        out_specs=pl.BlockSpec((None, Q, 256), lambda b, i: (b, i, 0)),
        out_shape=jax.ShapeDtypeStruct((B, N, 256), jnp.float32),
    )(at6, Wout, bout2)


def kernel(in_feats, sample_priors, sample_feats, map_hw, map_offs, map_ids,
           W_off, b_off, W_aw, b_aw, W_val, b_val, W_out, b_out):
    Wo = W_off.reshape(256, H, 8, 3)
    bo = b_off.reshape(H, 8, 3)
    Wbig = jnp.concatenate([Wo[..., 0].reshape(256, 64), Wo[..., 1].reshape(256, 64),
                            Wo[..., 2].reshape(256, 64), W_aw], axis=1)
    bbig = jnp.concatenate([bo[..., 0].reshape(64), bo[..., 1].reshape(64),
                            bo[..., 2].reshape(64), b_aw])
    G = jnp.asarray(np.kron(np.eye(8, dtype=np.float32),
                            np.ones((8, 8), dtype=np.float32)))

    inT = in_feats.reshape(B, NG, Q, 256).transpose(0, 1, 3, 2)
    priorT = sample_priors.reshape(B, NG, Q, 2).transpose(0, 1, 3, 2)
    midT = map_ids.astype(jnp.int32).reshape(B, NG, 1, Q)

    val5, idx4, w4 = _run_prep(
        inT, sample_feats, priorT, midT, Wbig.T, bbig[:, None],
        W_val, b_val[None, :], G, map_hw, map_offs)

    val_r = jnp.pad(val5.transpose(0, 1, 3, 2),
                    ((0, 0), (0, 0), (0, 0), (0, SP - S))
                    ).reshape(B, 16, 16 * SP)
    idx_c = idx4.reshape(2 * H, NG, 64 * Q)
    w_c = w4.reshape(2 * H, NG, 64 * Q)

    at5 = _sc_gather_call(val_r, idx_c, w_c)
    at6 = at5.reshape(B, NG, 16, 16, Q)

    return _run_outproj(at6, W_out, b_out[None, :])


# speedup vs baseline: 151.5507x; 1.0022x over previous
import functools
import math

import jax
import jax.numpy as jnp
import numpy as np
from jax import lax
from jax.experimental import pallas as pl
from jax.experimental.pallas import tpu as pltpu
from jax.experimental.pallas import tpu_sc as plsc

B, N, S = 2, 5440, 5440
H = 8
HD = 32
Q = 160
NG = N // Q
SP = 5504


def _prep_body(inT_ref, sf_ref, priorT_ref, midT_ref, WbigT_ref, bbigT_ref,
               Wval_ref, bval_ref, G_ref, hw_ref, offs_ref,
               val_ref, idx_ref, w_ref):
    f32 = jnp.float32
    xinT = inT_ref[...]
    sfx = sf_ref[...]

    val = jnp.dot(sfx, Wval_ref[...], preferred_element_type=f32) + bval_ref[...]
    for cg in range(16):
        val_ref[cg, :, :] = val[:, cg * 16:(cg + 1) * 16]

    big = jnp.dot(WbigT_ref[...], xinT, preferred_element_type=f32) + bbigT_ref[...]
    ox = big[0:64, :]
    oy = big[64:128, :]
    oz = big[128:192, :]
    aw = big[192:256, :]

    e = jnp.exp(aw)
    ssum = jnp.dot(G_ref[...], e, preferred_element_type=f32)
    attn = e / ssum

    mid = midT_ref[...]
    midf = mid.astype(f32)

    def sel4(lvl, t0, t1, t2, t3):
        return jnp.where(lvl == 0, t0, jnp.where(lvl == 1, t1,
                         jnp.where(lvl == 2, t2, t3)))

    hw = [(hw_ref[k, 0], hw_ref[k, 1]) for k in range(4)]
    offs_t = [offs_ref[k] for k in range(4)]

    normW = sel4(mid, hw[0][1], hw[1][1], hw[2][1], hw[3][1]).astype(f32)
    normH = sel4(mid, hw[0][0], hw[1][0], hw[2][0], hw[3][0]).astype(f32)
    x = priorT_ref[0:1, :] + ox / normW
    y = priorT_ref[1:2, :] + oy / normH

    pmod = (lax.broadcasted_iota(jnp.int32, (64, Q), 0) % 2).astype(f32)
    zbase = midf + (pmod - 0.5)
    z = (zbase + jnp.tanh(oz)) / 3.0
    zc = jnp.clip(z, 0.0, 1.0) * 3.0
    z0f = jnp.clip(jnp.floor(zc), 0.0, 2.0)
    wz = jnp.clip(zc - z0f, 0.0, 1.0)
    z0 = z0f.astype(jnp.int32)

    idx_blocks = []
    w_blocks = []
    for L in (0, 1):
        lvl = z0 + L
        Hl = sel4(lvl, hw[0][0], hw[1][0], hw[2][0], hw[3][0]).astype(f32)
        Wl = sel4(lvl, hw[0][1], hw[1][1], hw[2][1], hw[3][1]).astype(f32)
        offv = sel4(lvl, offs_t[0], offs_t[1], offs_t[2], offs_t[3])
        h_im = y * Hl - 0.5
        w_im = x * Wl - 0.5
        h0 = jnp.floor(h_im)
        w0 = jnp.floor(w_im)
        lh = h_im - h0
        lw = w_im - w0
        h0i = h0.astype(jnp.int32)
        w0i = w0.astype(jnp.int32)
        Hli = Hl.astype(jnp.int32)
        Wli = Wl.astype(jnp.int32)
        lvlw = (1.0 - wz) if L == 0 else wz
        base_w = attn * lvlw
        for dh, dw, cw in ((0, 0, (1 - lh) * (1 - lw)), (0, 1, (1 - lh) * lw),
                           (1, 0, lh * (1 - lw)), (1, 1, lh * lw)):
            hh = h0i + dh
            ww = w0i + dw
            valid = (hh >= 0) & (hh < Hli) & (ww >= 0) & (ww < Wli)
            hs = jnp.clip(hh, 0, Hli - 1)
            ws = jnp.clip(ww, 0, Wli - 1)
            idx_blocks.append(offv + hs * Wli + ws)
            w_blocks.append(jnp.where(valid, base_w * cw, 0.0))

    for h in range(8):
        for cb in range(8):
            idx_ref[h, cb * 8:(cb + 1) * 8, :] = idx_blocks[cb][h * 8:(h + 1) * 8, :]
            w_ref[h, cb * 8:(cb + 1) * 8, :] = w_blocks[cb][h * 8:(h + 1) * 8, :]


def _run_prep(inT, sf, priorT, midT, WbigT, bbigT, Wval, bval2, G, map_hw, map_offs):
    full = lambda shp: pl.BlockSpec(shp, lambda b, i: (0,) * len(shp))
    smem = lambda shp: pl.BlockSpec(shp, lambda b, i: (0,) * len(shp),
                                    memory_space=pltpu.SMEM)
    return pl.pallas_call(
        _prep_body,
        grid=(B, NG),
        in_specs=[pl.BlockSpec((None, None, 256, Q), lambda b, i: (b, i, 0, 0)),
                  pl.BlockSpec((None, Q, 256), lambda b, i: (b, i, 0)),
                  pl.BlockSpec((None, None, 2, Q), lambda b, i: (b, i, 0, 0)),
                  pl.BlockSpec((None, None, 1, Q), lambda b, i: (b, i, 0, 0)),
                  full((256, 256)), full((256, 1)),
                  full((256, 256)), full((1, 256)),
                  full((64, 64)), smem((4, 2)), smem((4,))],
        out_specs=[pl.BlockSpec((None, 16, Q, 16), lambda b, i: (b, 0, i, 0)),
                   pl.BlockSpec((8, None, 64, Q), lambda b, i: (b, i, 0, 0)),
                   pl.BlockSpec((8, None, 64, Q), lambda b, i: (b, i, 0, 0))],
        out_shape=[jax.ShapeDtypeStruct((B, 16, S, 16), jnp.float32),
                   jax.ShapeDtypeStruct((2 * H, NG, 64, Q), jnp.int32),
                   jax.ShapeDtypeStruct((2 * H, NG, 64, Q), jnp.float32)],
    )(inT, sf, priorT, midT, WbigT, bbigT, Wval, bval2, G, map_hw, map_offs)


def _sc_gather_call(val_r, idx_c, w_c):
    mesh = plsc.VectorSubcoreMesh(core_axis_name="c", subcore_axis_name="s")

    @functools.partial(
        pl.kernel,
        out_type=jax.ShapeDtypeStruct((B, NG, 16, 16 * Q), jnp.float32),
        mesh=mesh,
        compiler_params=pltpu.CompilerParams(needs_layout_passes=False),
        scratch_types=[
            pltpu.VMEM((16 * SP,), jnp.float32),
            pltpu.VMEM((64 * Q,), jnp.int32),
            pltpu.VMEM((64 * Q,), jnp.float32),
            pltpu.VMEM((16 * Q,), jnp.float32),
        ],
    )
    def sc_gather(val_hbm, idxt_hbm, wt_hbm, out_hbm, table_v, idx_v, w_v, ob_v):
        cid = lax.axis_index("c")
        sid = lax.axis_index("s")
        wid = sid * 2 + cid
        bh = wid % 16
        half = wid // 16
        b = bh // 8
        cg = (bh % 8) * 2 + half
        pltpu.sync_copy(val_hbm.at[b, cg], table_v)

        def chunk(gi, carry):
            pltpu.sync_copy(idxt_hbm.at[bh, gi], idx_v)
            pltpu.sync_copy(wt_hbm.at[bh, gi], w_v)

            @plsc.parallel_loop(0, Q // 16, unroll=2)
            def group(qg):
                qo = pl.multiple_of(qg * 16, 16)

                def jbody(j, acc):
                    iv = idx_v[pl.ds(j * Q + qo, 16)]
                    wv = w_v[pl.ds(j * Q + qo, 16)]
                    return tuple(
                        acc[d] + wv * plsc.load_gather(
                            table_v.at[pl.ds(d * SP, S)], [iv])
                        for d in range(16))

                acc = lax.fori_loop(
                    0, 64, jbody,
                    tuple(jnp.zeros((16,), jnp.float32) for _ in range(16)))
                for d in range(16):
                    ob_v[pl.ds(d * Q + qo, 16)] = acc[d]

            pltpu.sync_copy(ob_v, out_hbm.at[b, gi, cg])
            return carry

        lax.fori_loop(0, NG, chunk, 0)

    return sc_gather(val_r, idx_c, w_c)


def _outproj_body(at_ref, Wout_ref, bout_ref, out_ref):
    a = at_ref[...].reshape(256, Q)
    o = lax.dot_general(a, Wout_ref[...], (((0,), (0,)), ((), ())),
                        preferred_element_type=jnp.float32)
    out_ref[...] = o + bout_ref[...]


def _run_outproj(at6, Wout, bout2):
    full = lambda shp: pl.BlockSpec(shp, lambda b, i: (0,) * len(shp))
    return pl.pallas_call(
        _outproj_body,
        grid=(B, NG),
        in_specs=[pl.BlockSpec((None, None, 16, 16, Q),
                               lambda b, i: (b, i, 0, 0, 0)),
                  full((256, 256)), full((1, 256))],
        out_specs=pl.BlockSpec((None, Q, 256), lambda b, i: (b, i, 0)),
        out_shape=jax.ShapeDtypeStruct((B, N, 256), jnp.float32),
    )(at6, Wout, bout2)


def kernel(in_feats, sample_priors, sample_feats, map_hw, map_offs, map_ids,
           W_off, b_off, W_aw, b_aw, W_val, b_val, W_out, b_out):
    Wo = W_off.reshape(256, H, 8, 3)
    bo = b_off.reshape(H, 8, 3)
    Wbig = jnp.concatenate([Wo[..., 0].reshape(256, 64), Wo[..., 1].reshape(256, 64),
                            Wo[..., 2].reshape(256, 64), W_aw], axis=1)
    bbig = jnp.concatenate([bo[..., 0].reshape(64), bo[..., 1].reshape(64),
                            bo[..., 2].reshape(64), b_aw])
    G = jnp.asarray(np.kron(np.eye(8, dtype=np.float32),
                            np.ones((8, 8), dtype=np.float32)))

    inT = in_feats.reshape(B, NG, Q, 256).transpose(0, 1, 3, 2)
    priorT = sample_priors.reshape(B, NG, Q, 2).transpose(0, 1, 3, 2)
    midT = map_ids.astype(jnp.int32).reshape(B, NG, 1, Q)

    val5, idx4, w4 = _run_prep(
        inT, sample_feats, priorT, midT, Wbig.T, bbig[:, None],
        W_val, b_val[None, :], G, map_hw, map_offs)

    val_r = jnp.pad(val5.transpose(0, 1, 3, 2),
                    ((0, 0), (0, 0), (0, 0), (0, SP - S))
                    ).reshape(B, 16, 16 * SP)
    idx_c = idx4.reshape(2 * H, NG, 64 * Q)
    w_c = w4.reshape(2 * H, NG, 64 * Q)

    at5 = _sc_gather_call(val_r, idx_c, w_c)
    at6 = at5.reshape(B, NG, 16, 16, Q)

    return _run_outproj(at6, W_out, b_out[None, :])


# speedup vs baseline: 153.9994x; 1.0162x over previous
import functools
import math

import jax
import jax.numpy as jnp
import numpy as np
from jax import lax
from jax.experimental import pallas as pl
from jax.experimental.pallas import tpu as pltpu
from jax.experimental.pallas import tpu_sc as plsc

B, N, S = 2, 5440, 5440
H = 8
HD = 32
Q = 160
NG = N // Q
SP = 5504


def _prep_body(inT_ref, sf_ref, priorT_ref, midT_ref, WbigT_ref, bbigT_ref,
               Wval_ref, bval_ref, G_ref, hw_ref, offs_ref,
               val_ref, idx_ref, w_ref):
    f32 = jnp.float32
    xinT = inT_ref[...]
    sfx = sf_ref[...]

    val = jnp.dot(sfx, Wval_ref[...], preferred_element_type=f32) + bval_ref[...]
    for cg in range(16):
        val_ref[cg, :, :] = val[:, cg * 16:(cg + 1) * 16]

    big = jnp.dot(WbigT_ref[...], xinT, preferred_element_type=f32) + bbigT_ref[...]
    ox = big[0:64, :]
    oy = big[64:128, :]
    oz = big[128:192, :]
    aw = big[192:256, :]

    e = jnp.exp(aw)
    ssum = jnp.dot(G_ref[...], e, preferred_element_type=f32)
    attn = e / ssum

    mid = midT_ref[...]
    midf = mid.astype(f32)

    def sel4(lvl, t0, t1, t2, t3):
        return jnp.where(lvl == 0, t0, jnp.where(lvl == 1, t1,
                         jnp.where(lvl == 2, t2, t3)))

    hw = [(hw_ref[k, 0], hw_ref[k, 1]) for k in range(4)]
    offs_t = [offs_ref[k] for k in range(4)]

    normW = sel4(mid, hw[0][1], hw[1][1], hw[2][1], hw[3][1]).astype(f32)
    normH = sel4(mid, hw[0][0], hw[1][0], hw[2][0], hw[3][0]).astype(f32)
    x = priorT_ref[0:1, :] + ox / normW
    y = priorT_ref[1:2, :] + oy / normH

    pmod = (lax.broadcasted_iota(jnp.int32, (64, Q), 0) % 2).astype(f32)
    zbase = midf + (pmod - 0.5)
    z = (zbase + jnp.tanh(oz)) / 3.0
    zc = jnp.clip(z, 0.0, 1.0) * 3.0
    z0f = jnp.clip(jnp.floor(zc), 0.0, 2.0)
    wz = jnp.clip(zc - z0f, 0.0, 1.0)
    z0 = z0f.astype(jnp.int32)

    idx_blocks = []
    w_blocks = []
    for L in (0, 1):
        lvl = z0 + L
        Hl = sel4(lvl, hw[0][0], hw[1][0], hw[2][0], hw[3][0]).astype(f32)
        Wl = sel4(lvl, hw[0][1], hw[1][1], hw[2][1], hw[3][1]).astype(f32)
        offv = sel4(lvl, offs_t[0], offs_t[1], offs_t[2], offs_t[3])
        h_im = y * Hl - 0.5
        w_im = x * Wl - 0.5
        h0 = jnp.floor(h_im)
        w0 = jnp.floor(w_im)
        lh = h_im - h0
        lw = w_im - w0
        h0i = h0.astype(jnp.int32)
        w0i = w0.astype(jnp.int32)
        Hli = Hl.astype(jnp.int32)
        Wli = Wl.astype(jnp.int32)
        lvlw = (1.0 - wz) if L == 0 else wz
        base_w = attn * lvlw
        for dh, dw, cw in ((0, 0, (1 - lh) * (1 - lw)), (0, 1, (1 - lh) * lw),
                           (1, 0, lh * (1 - lw)), (1, 1, lh * lw)):
            hh = h0i + dh
            ww = w0i + dw
            valid = (hh >= 0) & (hh < Hli) & (ww >= 0) & (ww < Wli)
            hs = jnp.clip(hh, 0, Hli - 1)
            ws = jnp.clip(ww, 0, Wli - 1)
            idx_blocks.append(offv + hs * Wli + ws)
            w_blocks.append(jnp.where(valid, base_w * cw, 0.0))

    for h in range(8):
        for cb in range(8):
            idx_ref[h, cb * 8:(cb + 1) * 8, :] = idx_blocks[cb][h * 8:(h + 1) * 8, :]
            w_ref[h, cb * 8:(cb + 1) * 8, :] = w_blocks[cb][h * 8:(h + 1) * 8, :]


def _run_prep(inT, sf, priorT, midT, WbigT, bbigT, Wval, bval2, G, map_hw, map_offs):
    full = lambda shp: pl.BlockSpec(shp, lambda b, i: (0,) * len(shp))
    smem = lambda shp: pl.BlockSpec(shp, lambda b, i: (0,) * len(shp),
                                    memory_space=pltpu.SMEM)
    return pl.pallas_call(
        _prep_body,
        grid=(B, NG),
        in_specs=[pl.BlockSpec((None, None, 256, Q), lambda b, i: (b, i, 0, 0)),
                  pl.BlockSpec((None, Q, 256), lambda b, i: (b, i, 0)),
                  pl.BlockSpec((None, None, 2, Q), lambda b, i: (b, i, 0, 0)),
                  pl.BlockSpec((None, None, 1, Q), lambda b, i: (b, i, 0, 0)),
                  full((256, 256)), full((256, 1)),
                  full((256, 256)), full((1, 256)),
                  full((64, 64)), smem((4, 2)), smem((4,))],
        out_specs=[pl.BlockSpec((None, 16, Q, 16), lambda b, i: (b, 0, i, 0)),
                   pl.BlockSpec((8, None, 64, Q), lambda b, i: (b, i, 0, 0)),
                   pl.BlockSpec((8, None, 64, Q), lambda b, i: (b, i, 0, 0))],
        out_shape=[jax.ShapeDtypeStruct((B, 16, S, 16), jnp.float32),
                   jax.ShapeDtypeStruct((2 * H, NG, 64, Q), jnp.int32),
                   jax.ShapeDtypeStruct((2 * H, NG, 64, Q), jnp.float32)],
    )(inT, sf, priorT, midT, WbigT, bbigT, Wval, bval2, G, map_hw, map_offs)


def _sc_gather_call(val_r, idx_c, w_c):
    mesh = plsc.VectorSubcoreMesh(core_axis_name="c", subcore_axis_name="s")

    @functools.partial(
        pl.kernel,
        out_type=jax.ShapeDtypeStruct((B, NG, 16, 16 * Q), jnp.float32),
        mesh=mesh,
        compiler_params=pltpu.CompilerParams(needs_layout_passes=False),
        scratch_types=[
            pltpu.VMEM((16 * SP,), jnp.float32),
            pltpu.VMEM((64 * Q,), jnp.int32),
            pltpu.VMEM((64 * Q,), jnp.float32),
            pltpu.VMEM((16 * Q,), jnp.float32),
        ],
    )
    def sc_gather(val_hbm, idxt_hbm, wt_hbm, out_hbm, table_v, idx_v, w_v, ob_v):
        cid = lax.axis_index("c")
        sid = lax.axis_index("s")
        wid = sid * 2 + cid
        bh = wid % 16
        half = wid // 16
        b = bh // 8
        cg = (bh % 8) * 2 + half
        pltpu.sync_copy(val_hbm.at[b, cg], table_v)

        def chunk(gi, carry):
            pltpu.sync_copy(idxt_hbm.at[bh, gi], idx_v)
            pltpu.sync_copy(wt_hbm.at[bh, gi], w_v)

            @plsc.parallel_loop(0, Q // 16, unroll=2)
            def group(qg):
                qo = pl.multiple_of(qg * 16, 16)

                def jbody(j, acc):
                    iv = idx_v[pl.ds(j * Q + qo, 16)]
                    wv = w_v[pl.ds(j * Q + qo, 16)]
                    return tuple(
                        acc[d] + wv * plsc.load_gather(
                            table_v.at[pl.ds(d * SP, S)], [iv])
                        for d in range(16))

                acc = lax.fori_loop(
                    0, 64, jbody,
                    tuple(jnp.zeros((16,), jnp.float32) for _ in range(16)),
                    unroll=2)
                for d in range(16):
                    ob_v[pl.ds(d * Q + qo, 16)] = acc[d]

            pltpu.sync_copy(ob_v, out_hbm.at[b, gi, cg])
            return carry

        lax.fori_loop(0, NG, chunk, 0)

    return sc_gather(val_r, idx_c, w_c)


def _outproj_body(at_ref, Wout_ref, bout_ref, out_ref):
    a = at_ref[...].reshape(256, Q)
    o = lax.dot_general(a, Wout_ref[...], (((0,), (0,)), ((), ())),
                        preferred_element_type=jnp.float32)
    out_ref[...] = o + bout_ref[...]


def _run_outproj(at6, Wout, bout2):
    full = lambda shp: pl.BlockSpec(shp, lambda b, i: (0,) * len(shp))
    return pl.pallas_call(
        _outproj_body,
        grid=(B, NG),
        in_specs=[pl.BlockSpec((None, None, 16, 16, Q),
                               lambda b, i: (b, i, 0, 0, 0)),
                  full((256, 256)), full((1, 256))],
        out_specs=pl.BlockSpec((None, Q, 256), lambda b, i: (b, i, 0)),
        out_shape=jax.ShapeDtypeStruct((B, N, 256), jnp.float32),
    )(at6, Wout, bout2)


def kernel(in_feats, sample_priors, sample_feats, map_hw, map_offs, map_ids,
           W_off, b_off, W_aw, b_aw, W_val, b_val, W_out, b_out):
    Wo = W_off.reshape(256, H, 8, 3)
    bo = b_off.reshape(H, 8, 3)
    Wbig = jnp.concatenate([Wo[..., 0].reshape(256, 64), Wo[..., 1].reshape(256, 64),
                            Wo[..., 2].reshape(256, 64), W_aw], axis=1)
    bbig = jnp.concatenate([bo[..., 0].reshape(64), bo[..., 1].reshape(64),
                            bo[..., 2].reshape(64), b_aw])
    G = jnp.asarray(np.kron(np.eye(8, dtype=np.float32),
                            np.ones((8, 8), dtype=np.float32)))

    inT = in_feats.reshape(B, NG, Q, 256).transpose(0, 1, 3, 2)
    priorT = sample_priors.reshape(B, NG, Q, 2).transpose(0, 1, 3, 2)
    midT = map_ids.astype(jnp.int32).reshape(B, NG, 1, Q)

    val5, idx4, w4 = _run_prep(
        inT, sample_feats, priorT, midT, Wbig.T, bbig[:, None],
        W_val, b_val[None, :], G, map_hw, map_offs)

    val_r = jnp.pad(val5.transpose(0, 1, 3, 2),
                    ((0, 0), (0, 0), (0, 0), (0, SP - S))
                    ).reshape(B, 16, 16 * SP)
    idx_c = idx4.reshape(2 * H, NG, 64 * Q)
    w_c = w4.reshape(2 * H, NG, 64 * Q)

    at5 = _sc_gather_call(val_r, idx_c, w_c)
    at6 = at5.reshape(B, NG, 16, 16, Q)

    return _run_outproj(at6, W_out, b_out[None, :])


# speedup vs baseline: 155.5627x; 1.0102x over previous
import functools
import math

import jax
import jax.numpy as jnp
import numpy as np
from jax import lax
from jax.experimental import pallas as pl
from jax.experimental.pallas import tpu as pltpu
from jax.experimental.pallas import tpu_sc as plsc

B, N, S = 2, 5440, 5440
H = 8
HD = 32
Q = 160
NG = N // Q
SP = 5504


def _valproj_body(sf_ref, Wval_ref, bval_ref, val_ref):
    f32 = jnp.float32
    sfx = sf_ref[...]
    val = jnp.dot(sfx, Wval_ref[...], preferred_element_type=f32) + bval_ref[...]
    for cg in range(16):
        val_ref[cg, :, :] = val[:, cg * 16:(cg + 1) * 16]


def _run_valproj(sf, Wval, bval2):
    full = lambda shp: pl.BlockSpec(shp, lambda b, i: (0,) * len(shp))
    return pl.pallas_call(
        _valproj_body,
        grid=(B, NG),
        in_specs=[pl.BlockSpec((None, Q, 256), lambda b, i: (b, i, 0)),
                  full((256, 256)), full((1, 256))],
        out_specs=pl.BlockSpec((None, 16, Q, 16), lambda b, i: (b, 0, i, 0)),
        out_shape=jax.ShapeDtypeStruct((B, 16, S, 16), jnp.float32),
    )(sf, Wval, bval2)


def _prep_body(inT_ref, priorT_ref, midT_ref, WbigT_ref, bbigT_ref,
               G_ref, hw_ref, offs_ref, idx_ref, w_ref):
    f32 = jnp.float32
    xinT = inT_ref[...]

    big = jnp.dot(WbigT_ref[...], xinT, preferred_element_type=f32) + bbigT_ref[...]
    ox = big[0:64, :]
    oy = big[64:128, :]
    oz = big[128:192, :]
    aw = big[192:256, :]

    e = jnp.exp(aw)
    ssum = jnp.dot(G_ref[...], e, preferred_element_type=f32)
    attn = e / ssum

    mid = midT_ref[...]
    midf = mid.astype(f32)

    def sel4(lvl, t0, t1, t2, t3):
        return jnp.where(lvl == 0, t0, jnp.where(lvl == 1, t1,
                         jnp.where(lvl == 2, t2, t3)))

    hw = [(hw_ref[k, 0], hw_ref[k, 1]) for k in range(4)]
    offs_t = [offs_ref[k] for k in range(4)]

    normW = sel4(mid, hw[0][1], hw[1][1], hw[2][1], hw[3][1]).astype(f32)
    normH = sel4(mid, hw[0][0], hw[1][0], hw[2][0], hw[3][0]).astype(f32)
    x = priorT_ref[0:1, :] + ox / normW
    y = priorT_ref[1:2, :] + oy / normH

    pmod = (lax.broadcasted_iota(jnp.int32, (64, Q), 0) % 2).astype(f32)
    zbase = midf + (pmod - 0.5)
    z = (zbase + jnp.tanh(oz)) / 3.0
    zc = jnp.clip(z, 0.0, 1.0) * 3.0
    z0f = jnp.clip(jnp.floor(zc), 0.0, 2.0)
    wz = jnp.clip(zc - z0f, 0.0, 1.0)
    z0 = z0f.astype(jnp.int32)

    idx_blocks = []
    w_blocks = []
    for L in (0, 1):
        lvl = z0 + L
        Hl = sel4(lvl, hw[0][0], hw[1][0], hw[2][0], hw[3][0]).astype(f32)
        Wl = sel4(lvl, hw[0][1], hw[1][1], hw[2][1], hw[3][1]).astype(f32)
        offv = sel4(lvl, offs_t[0], offs_t[1], offs_t[2], offs_t[3])
        h_im = y * Hl - 0.5
        w_im = x * Wl - 0.5
        h0 = jnp.floor(h_im)
        w0 = jnp.floor(w_im)
        lh = h_im - h0
        lw = w_im - w0
        h0i = h0.astype(jnp.int32)
        w0i = w0.astype(jnp.int32)
        Hli = Hl.astype(jnp.int32)
        Wli = Wl.astype(jnp.int32)
        lvlw = (1.0 - wz) if L == 0 else wz
        base_w = attn * lvlw
        for dh, dw, cw in ((0, 0, (1 - lh) * (1 - lw)), (0, 1, (1 - lh) * lw),
                           (1, 0, lh * (1 - lw)), (1, 1, lh * lw)):
            hh = h0i + dh
            ww = w0i + dw
            valid = (hh >= 0) & (hh < Hli) & (ww >= 0) & (ww < Wli)
            hs = jnp.clip(hh, 0, Hli - 1)
            ws = jnp.clip(ww, 0, Wli - 1)
            idx_blocks.append(offv + hs * Wli + ws)
            w_blocks.append(jnp.where(valid, base_w * cw, 0.0))

    for h in range(8):
        for cb in range(8):
            idx_ref[h, cb * 8:(cb + 1) * 8, :] = idx_blocks[cb][h * 8:(h + 1) * 8, :]
            w_ref[h, cb * 8:(cb + 1) * 8, :] = w_blocks[cb][h * 8:(h + 1) * 8, :]


def _run_prep(inT, priorT, midT, WbigT, bbigT, G, map_hw, map_offs, off, ngh):
    full = lambda shp: pl.BlockSpec(shp, lambda b, i: (0,) * len(shp))
    smem = lambda shp: pl.BlockSpec(shp, lambda b, i: (0,) * len(shp),
                                    memory_space=pltpu.SMEM)
    return pl.pallas_call(
        _prep_body,
        grid=(B, ngh),
        in_specs=[pl.BlockSpec((None, None, 256, Q), lambda b, i: (b, i + off, 0, 0)),
                  pl.BlockSpec((None, None, 2, Q), lambda b, i: (b, i + off, 0, 0)),
                  pl.BlockSpec((None, None, 1, Q), lambda b, i: (b, i + off, 0, 0)),
                  full((256, 256)), full((256, 1)),
                  full((64, 64)), smem((4, 2)), smem((4,))],
        out_specs=[pl.BlockSpec((8, None, 64, Q), lambda b, i: (b, i, 0, 0)),
                   pl.BlockSpec((8, None, 64, Q), lambda b, i: (b, i, 0, 0))],
        out_shape=[jax.ShapeDtypeStruct((2 * H, ngh, 64, Q), jnp.int32),
                   jax.ShapeDtypeStruct((2 * H, ngh, 64, Q), jnp.float32)],
    )(inT, priorT, midT, WbigT, bbigT, G, map_hw, map_offs)


def _sc_gather_call(val_r, idx_c, w_c, ngh):
    mesh = plsc.VectorSubcoreMesh(core_axis_name="c", subcore_axis_name="s")

    @functools.partial(
        pl.kernel,
        out_type=jax.ShapeDtypeStruct((B, ngh, 16, 16 * Q), jnp.float32),
        mesh=mesh,
        compiler_params=pltpu.CompilerParams(needs_layout_passes=False),
        scratch_types=[
            pltpu.VMEM((16 * SP,), jnp.float32),
            pltpu.VMEM((64 * Q,), jnp.int32),
            pltpu.VMEM((64 * Q,), jnp.float32),
            pltpu.VMEM((16 * Q,), jnp.float32),
        ],
    )
    def sc_gather(val_hbm, idxt_hbm, wt_hbm, out_hbm, table_v, idx_v, w_v, ob_v):
        cid = lax.axis_index("c")
        sid = lax.axis_index("s")
        wid = sid * 2 + cid
        bh = wid % 16
        half = wid // 16
        b = bh // 8
        cg = (bh % 8) * 2 + half
        pltpu.sync_copy(val_hbm.at[b, cg], table_v)

        def chunk(gi, carry):
            pltpu.sync_copy(idxt_hbm.at[bh, gi], idx_v)
            pltpu.sync_copy(wt_hbm.at[bh, gi], w_v)

            @plsc.parallel_loop(0, Q // 16, unroll=2)
            def group(qg):
                qo = pl.multiple_of(qg * 16, 16)

                def jbody(j, acc):
                    iv = idx_v[pl.ds(j * Q + qo, 16)]
                    wv = w_v[pl.ds(j * Q + qo, 16)]
                    return tuple(
                        acc[d] + wv * plsc.load_gather(
                            table_v.at[pl.ds(d * SP, S)], [iv])
                        for d in range(16))

                acc = lax.fori_loop(
                    0, 64, jbody,
                    tuple(jnp.zeros((16,), jnp.float32) for _ in range(16)),
                    unroll=2)
                for d in range(16):
                    ob_v[pl.ds(d * Q + qo, 16)] = acc[d]

            pltpu.sync_copy(ob_v, out_hbm.at[b, gi, cg])
            return carry

        lax.fori_loop(0, ngh, chunk, 0)

    return sc_gather(val_r, idx_c, w_c)


def _outproj_body(at_ref, Wout_ref, bout_ref, out_ref):
    a = at_ref[...].reshape(256, Q)
    o = lax.dot_general(a, Wout_ref[...], (((0,), (0,)), ((), ())),
                        preferred_element_type=jnp.float32)
    out_ref[...] = o + bout_ref[...]


def _run_outproj(at6, Wout, bout2, ngh):
    full = lambda shp: pl.BlockSpec(shp, lambda b, i: (0,) * len(shp))
    return pl.pallas_call(
        _outproj_body,
        grid=(B, ngh),
        in_specs=[pl.BlockSpec((None, None, 16, 16, Q),
                               lambda b, i: (b, i, 0, 0, 0)),
                  full((256, 256)), full((1, 256))],
        out_specs=pl.BlockSpec((None, Q, 256), lambda b, i: (b, i, 0)),
        out_shape=jax.ShapeDtypeStruct((B, ngh * Q, 256), jnp.float32),
    )(at6, Wout, bout2)


def kernel(in_feats, sample_priors, sample_feats, map_hw, map_offs, map_ids,
           W_off, b_off, W_aw, b_aw, W_val, b_val, W_out, b_out):
    Wo = W_off.reshape(256, H, 8, 3)
    bo = b_off.reshape(H, 8, 3)
    Wbig = jnp.concatenate([Wo[..., 0].reshape(256, 64), Wo[..., 1].reshape(256, 64),
                            Wo[..., 2].reshape(256, 64), W_aw], axis=1)
    bbig = jnp.concatenate([bo[..., 0].reshape(64), bo[..., 1].reshape(64),
                            bo[..., 2].reshape(64), b_aw])
    G = jnp.asarray(np.kron(np.eye(8, dtype=np.float32),
                            np.ones((8, 8), dtype=np.float32)))

    inT = in_feats.reshape(B, NG, Q, 256).transpose(0, 1, 3, 2)
    priorT = sample_priors.reshape(B, NG, Q, 2).transpose(0, 1, 3, 2)
    midT = map_ids.astype(jnp.int32).reshape(B, NG, 1, Q)

    val5 = _run_valproj(sample_feats, W_val, b_val[None, :])
    val_r = jnp.pad(val5.transpose(0, 1, 3, 2),
                    ((0, 0), (0, 0), (0, 0), (0, SP - S))
                    ).reshape(B, 16, 16 * SP)

    ngh = NG // 2
    outs = []
    for off in (0, ngh):
        idx4, w4 = _run_prep(inT, priorT, midT, Wbig.T, bbig[:, None],
                             G, map_hw, map_offs, off, ngh)
        at5 = _sc_gather_call(val_r, idx4.reshape(2 * H, ngh, 64 * Q),
                              w4.reshape(2 * H, ngh, 64 * Q), ngh)
        at6 = at5.reshape(B, ngh, 16, 16, Q)
        outs.append(_run_outproj(at6, W_out, b_out[None, :], ngh))

    return jnp.concatenate(outs, axis=1)
